# Initial kernel scaffold; baseline (speedup 1.0000x reference)
import jax
import jax.numpy as jnp
from jax.experimental import pallas as pl


def kernel(token_ids, mapping):
    raise NotImplementedError("write your pallas kernel here")


# speedup vs baseline: 216.5065x; 216.5065x over previous
import functools

import jax
import jax.numpy as jnp
from jax import lax
from jax.experimental import pallas as pl
from jax.experimental.pallas import tpu as pltpu
from jax.experimental.pallas import tpu_sc as plsc

_B, _S = 16384, 200
_N = _B * _S
_VOCAB = 1_000_000

_NC, _NS = 2, 16
_NW = _NC * _NS
_PER_W = _N // _NW
_CH = 25_600
_NCHUNK = _PER_W // _CH

_TBL_CH = 62_496
_TBL_P = _TBL_CH // 3
_TBL_TAIL = _VOCAB - _NS * _TBL_CH

_mesh = plsc.VectorSubcoreMesh(core_axis_name="c", subcore_axis_name="s")


@functools.partial(
    pl.kernel,
    mesh=_mesh,
    out_type=jax.ShapeDtypeStruct((_N,), jnp.int32),
    scratch_types=[
        pltpu.VMEM_SHARED((_VOCAB,), jnp.int32),
        pltpu.VMEM((_CH,), jnp.int32),
        pltpu.VMEM((_CH,), jnp.int32),
        pltpu.SemaphoreType.DMA,
    ],
)
def _lookup(ids_hbm, map_hbm, out_hbm, tbl_sh, idx_v, val_v, sem):
    cid = lax.axis_index("c")
    sid = lax.axis_index("s")
    wid = sid * _NC + cid

    for p in range(3):
        toff = sid * _TBL_CH + p * _TBL_P
        pltpu.sync_copy(map_hbm.at[pl.ds(toff, _TBL_P)], val_v.at[pl.ds(0, _TBL_P)])
        pltpu.sync_copy(val_v.at[pl.ds(0, _TBL_P)], tbl_sh.at[pl.ds(toff, _TBL_P)])

    @pl.when(sid == _NS - 1)
    def _copy_tail():
        pltpu.sync_copy(
            map_hbm.at[pl.ds(_NS * _TBL_CH, _TBL_TAIL)],
            val_v.at[pl.ds(0, _TBL_TAIL)],
        )
        pltpu.sync_copy(
            val_v.at[pl.ds(0, _TBL_TAIL)],
            tbl_sh.at[pl.ds(_NS * _TBL_CH, _TBL_TAIL)],
        )

    plsc.subcore_barrier()

    base = wid * _PER_W
    for k in range(_NCHUNK):
        off = base + k * _CH
        pltpu.sync_copy(ids_hbm.at[pl.ds(off, _CH)], idx_v)
        pltpu.async_copy(tbl_sh.at[idx_v], val_v, sem).wait()
        pltpu.sync_copy(val_v, out_hbm.at[pl.ds(off, _CH)])


def kernel(token_ids, mapping):
    out = _lookup(token_ids.reshape(_N), mapping)
    return out.reshape(token_ids.shape)


# speedup vs baseline: 230.0293x; 1.0625x over previous
import functools

import jax
import jax.numpy as jnp
from jax import lax
from jax.experimental import pallas as pl
from jax.experimental.pallas import tpu as pltpu
from jax.experimental.pallas import tpu_sc as plsc

_B, _S = 16384, 200
_N = _B * _S
_VOCAB = 1_000_000

_NC, _NS = 2, 16
_NW = _NC * _NS
_PER_W = _N // _NW
_CH = 12_800
_NCHUNK = _PER_W // _CH

_TBL_CH = 62_464
_TBL_P = _TBL_CH // 4
_TBL_TAIL = _VOCAB - _NS * _TBL_CH

_mesh = plsc.VectorSubcoreMesh(core_axis_name="c", subcore_axis_name="s")


@functools.partial(
    pl.kernel,
    mesh=_mesh,
    out_type=jax.ShapeDtypeStruct((_N,), jnp.int32),
    scratch_types=[
        pltpu.VMEM_SHARED((_VOCAB,), jnp.int32),
        pltpu.VMEM((_CH,), jnp.int32),
        pltpu.VMEM((_CH,), jnp.int32),
        pltpu.VMEM((_CH,), jnp.int32),
        pltpu.VMEM((_CH,), jnp.int32),
        pltpu.VMEM((_TBL_P,), jnp.int32),
        pltpu.SemaphoreType.DMA,
        pltpu.SemaphoreType.DMA,
        pltpu.SemaphoreType.DMA,
        pltpu.SemaphoreType.DMA,
        pltpu.SemaphoreType.DMA,
    ],
)
def _lookup(ids_hbm, map_hbm, out_hbm, tbl_sh, idx0, idx1, val0, val1, stg_v,
            si0, si1, sg, so0, so1):
    cid = lax.axis_index("c")
    sid = lax.axis_index("s")
    wid = sid * _NC + cid
    base = wid * _PER_W

    idx_b = (idx0, idx1)
    val_b = (val0, val1)
    sem_i = (si0, si1)
    sem_o = (so0, so1)

    idx_cp = [None] * _NCHUNK
    for k in range(2):
        idx_cp[k] = pltpu.async_copy(
            ids_hbm.at[pl.ds(base + k * _CH, _CH)], idx_b[k], sem_i[k])

    for p in range(4):
        toff = sid * _TBL_CH + p * _TBL_P
        pltpu.sync_copy(map_hbm.at[pl.ds(toff, _TBL_P)], stg_v)
        pltpu.sync_copy(stg_v, tbl_sh.at[pl.ds(toff, _TBL_P)])

    @pl.when(sid == _NS - 1)
    def _copy_tail():
        pltpu.sync_copy(
            map_hbm.at[pl.ds(_NS * _TBL_CH, _TBL_TAIL)],
            stg_v.at[pl.ds(0, _TBL_TAIL)],
        )
        pltpu.sync_copy(
            stg_v.at[pl.ds(0, _TBL_TAIL)],
            tbl_sh.at[pl.ds(_NS * _TBL_CH, _TBL_TAIL)],
        )

    plsc.subcore_barrier()

    out_cp = [None, None]
    for k in range(_NCHUNK):
        b = k % 2
        idx_cp[k].wait()
        if out_cp[b] is not None:
            out_cp[b].wait()
        pltpu.async_copy(tbl_sh.at[idx_b[b]], val_b[b], sg).wait()
        if k + 2 < _NCHUNK:
            idx_cp[k + 2] = pltpu.async_copy(
                ids_hbm.at[pl.ds(base + (k + 2) * _CH, _CH)],
                idx_b[b], sem_i[b])
        out_cp[b] = pltpu.async_copy(
            val_b[b], out_hbm.at[pl.ds(base + k * _CH, _CH)], sem_o[b])

    out_cp[0].wait()
    out_cp[1].wait()


def kernel(token_ids, mapping):
    out = _lookup(token_ids.reshape(_N), mapping)
    return out.reshape(token_ids.shape)
